# baseline (speedup 1.0000x reference)
import jax
import jax.numpy as jnp
from jax import lax
from jax.experimental import pallas as pl
from jax.experimental.pallas import tpu as pltpu
from jax.experimental.pallas import tpu_sc as plsc

_C = 1024
_S = 2048
_L = 4096
_NC = 2
_NS = 16
_NW = _NC * _NS
_NBLK = 16
_IPB = _S // _NBLK
_CHALF = _C // 2
_CROWS = _C // _NW
_RPC = _L // 128
_SH = _S // 128
_NCCH = 8


def _body(x_hbm, cache_hbm, emb_hbm, out_hbm, idx_v, rows_a, rows_b,
          trans_a, trans_b, cbuf_a, cbuf_b, tidx_a, tidx_b,
          gidx_a, gidx_b, sidx_a, sidx_b, gsem, tsem, cgsem, cssem):
    wid = lax.axis_index("s") * _NC + lax.axis_index("c")
    blk = wid // 2
    half = wid % 2
    coff = half * _CHALF
    crow = wid * _CROWS
    iota = lax.iota(jnp.int32, 16)
    rows = (rows_a, rows_b)
    trans = (trans_a, trans_b)
    cbuf = (cbuf_a, cbuf_b)
    tidx = (tidx_a, tidx_b)
    gidx = (gidx_a, gidx_b)
    sidx = (sidx_a, sidx_b)

    pltpu.sync_copy(x_hbm.at[pl.ds(blk * _IPB, _IPB)], idx_v)
    h_emb = [None, None]
    h_emb[0] = pltpu.async_copy(
        emb_hbm.at[idx_v, pl.ds(coff, _CHALF // 2)], rows_a, gsem)

    def cache_fire_gather(k):
        s = k % 2
        for cc in range(4):
            base = (crow + k * 4 + cc) * _RPC
            gidx[s][pl.ds(cc * 16, 16)] = base + _SH + iota
            sidx[s][pl.ds(cc * 16, 16)] = base + iota
        return pltpu.async_copy(cache_hbm.at[gidx[s]], cbuf[s], cgsem)

    h_cg = [None] * _NCCH
    h_cs = [None] * _NCCH
    h_cg[0] = cache_fire_gather(0)
    h_emb[1] = pltpu.async_copy(
        emb_hbm.at[idx_v, pl.ds(coff + _CHALF // 2, _CHALF // 2)],
        rows_b, gsem)

    def do_cache_step(k):
        h_cg[k].wait()
        h_cs[k] = pltpu.async_copy(
            cbuf[k % 2], out_hbm.at[sidx[k % 2]], cssem)
        if k + 1 < _NCCH:
            if k >= 1:
                h_cs[k - 1].wait()
            h_cg[k + 1] = cache_fire_gather(k + 1)

    do_cache_step(0)

    cache_sched = {0: (1,), 1: (2, 3), 2: (4, 5), 3: (6, 7)}

    h_ts = [None] * 4
    for q in range(4):
        s = q % 2
        if q in (0, 2):
            h_emb[q // 2].wait()
        if q >= 2:
            h_ts[q - 2].wait()

        @plsc.parallel_loop(0, 16)
        def tq(k, q=q, s=s):
            rot = jnp.bitwise_and(iota + k, 15)
            for ct in range(8):
                c_idx = ct * 16 + iota
                for jt in range(8):
                    j_idx = jt * 16 + rot
                    vals = plsc.load_gather(
                        rows[q // 2], [j_idx, (q % 2) * 128 + c_idx])
                    plsc.store_scatter(trans[s], [c_idx, j_idx], vals)

        for v in range(8):
            tidx[s][pl.ds(v * 16, 16)] = (
                (coff + q * 128 + v * 16 + iota) * _RPC + _SH + blk)
        h_ts[q] = pltpu.async_copy(trans[s], out_hbm.at[tidx[s]], tsem)

        for k in cache_sched[q]:
            do_cache_step(k)

    h_ts[2].wait()
    h_ts[3].wait()
    h_cs[_NCCH - 2].wait()
    h_cs[_NCCH - 1].wait()


@jax.jit
def _net(x_flat, cache2d, emb):
    mesh = plsc.VectorSubcoreMesh(core_axis_name="c", subcore_axis_name="s")
    return pl.kernel(
        _body,
        out_type=jax.ShapeDtypeStruct((_C * _L // 128, 128), jnp.float32),
        mesh=mesh,
        compiler_params=pltpu.CompilerParams(needs_layout_passes=False),
        scratch_types=[
            pltpu.VMEM((_IPB,), jnp.int32),
            pltpu.VMEM((_IPB, _CHALF // 2), jnp.float32),
            pltpu.VMEM((_IPB, _CHALF // 2), jnp.float32),
            pltpu.VMEM((128, 128), jnp.float32),
            pltpu.VMEM((128, 128), jnp.float32),
            pltpu.VMEM((64, 128), jnp.float32),
            pltpu.VMEM((64, 128), jnp.float32),
            pltpu.VMEM((128,), jnp.int32),
            pltpu.VMEM((128,), jnp.int32),
            pltpu.VMEM((64,), jnp.int32),
            pltpu.VMEM((64,), jnp.int32),
            pltpu.VMEM((64,), jnp.int32),
            pltpu.VMEM((64,), jnp.int32),
            pltpu.SemaphoreType.DMA,
            pltpu.SemaphoreType.DMA,
            pltpu.SemaphoreType.DMA,
            pltpu.SemaphoreType.DMA,
        ],
    )(x_flat, cache2d, emb)


def kernel(x, cache, emb):
    out2d = _net(x.reshape(_S), cache.reshape(_C * _L // 128, 128), emb)
    return out2d.reshape(1, _C, 1, _L)


# speedup vs baseline: 1.3531x; 1.3531x over previous
import jax
import jax.numpy as jnp
from jax import lax
from jax.experimental import pallas as pl
from jax.experimental.pallas import tpu as pltpu
from jax.experimental.pallas import tpu_sc as plsc

_C = 1024
_S = 2048
_L = 4096
_NC = 2
_NS = 16
_NW = _NC * _NS
_NBLK = 16
_IPB = _S // _NBLK
_CHALF = _C // 2
_CROWS = _C // _NW
_RPC = _L // 128
_SH = _S // 128
_NCCH = 8


def _body(x_hbm, cache_hbm, emb_hbm, out_hbm, idx_v, rows_a, rows_b,
          trans_a, trans_b, cbuf_a, cbuf_b, tidx_a, tidx_b,
          gidx_a, gidx_b, sidx_a, sidx_b, gsem, tsem, cgsem, cssem):
    wid = lax.axis_index("s") * _NC + lax.axis_index("c")
    blk = wid // 2
    half = wid % 2
    coff = half * _CHALF
    crow = wid * _CROWS
    iota = lax.iota(jnp.int32, 16)
    rows = (rows_a, rows_b)
    trans = (trans_a, trans_b)
    cbuf = (cbuf_a, cbuf_b)
    tidx = (tidx_a, tidx_b)
    gidx = (gidx_a, gidx_b)
    sidx = (sidx_a, sidx_b)

    pltpu.sync_copy(x_hbm.at[pl.ds(blk * _IPB, _IPB)], idx_v)
    h_emb = [None, None]
    h_emb[0] = pltpu.async_copy(
        emb_hbm.at[idx_v, pl.ds(coff, _CHALF // 2)], rows_a, gsem)

    def cache_fire_gather(k):
        s = k % 2
        for cc in range(4):
            base = (crow + k * 4 + cc) * _RPC
            gidx[s][pl.ds(cc * 16, 16)] = base + _SH + iota
            sidx[s][pl.ds(cc * 16, 16)] = base + iota
        return pltpu.async_copy(cache_hbm.at[gidx[s]], cbuf[s], cgsem)

    h_cg = [None] * _NCCH
    h_cs = [None] * _NCCH
    h_cg[0] = cache_fire_gather(0)
    h_emb[1] = pltpu.async_copy(
        emb_hbm.at[idx_v, pl.ds(coff + _CHALF // 2, _CHALF // 2)],
        rows_b, gsem)

    def do_cache_step(k):
        h_cg[k].wait()
        h_cs[k] = pltpu.async_copy(
            cbuf[k % 2], out_hbm.at[sidx[k % 2]], cssem)
        if k + 1 < _NCCH:
            if k >= 1:
                h_cs[k - 1].wait()
            h_cg[k + 1] = cache_fire_gather(k + 1)

    do_cache_step(0)

    cache_sched = {0: (1,), 1: (2, 3), 2: (4, 5), 3: (6, 7)}

    h_ts = [None] * 4
    for q in range(4):
        s = q % 2
        if q in (0, 2):
            h_emb[q // 2].wait()
        if q >= 2:
            h_ts[q - 2].wait()

        @plsc.parallel_loop(0, 128)
        def tq(t, q=q, s=s):
            j_idx = jnp.bitwise_and(t, -16) + jnp.bitwise_and(iota + t, 15)
            for ct in range(8):
                c_idx = ct * 16 + iota
                vals = plsc.load_gather(
                    rows[q // 2], [j_idx, (q % 2) * 128 + c_idx])
                plsc.store_scatter(trans[s], [c_idx, j_idx], vals)

        for v in range(8):
            tidx[s][pl.ds(v * 16, 16)] = (
                (coff + q * 128 + v * 16 + iota) * _RPC + _SH + blk)
        h_ts[q] = pltpu.async_copy(trans[s], out_hbm.at[tidx[s]], tsem)

        for k in cache_sched[q]:
            do_cache_step(k)

    h_ts[2].wait()
    h_ts[3].wait()
    h_cs[_NCCH - 2].wait()
    h_cs[_NCCH - 1].wait()


@jax.jit
def _net(x_flat, cache2d, emb):
    mesh = plsc.VectorSubcoreMesh(core_axis_name="c", subcore_axis_name="s")
    return pl.kernel(
        _body,
        out_type=jax.ShapeDtypeStruct((_C * _L // 128, 128), jnp.float32),
        mesh=mesh,
        compiler_params=pltpu.CompilerParams(needs_layout_passes=False),
        scratch_types=[
            pltpu.VMEM((_IPB,), jnp.int32),
            pltpu.VMEM((_IPB, _CHALF // 2), jnp.float32),
            pltpu.VMEM((_IPB, _CHALF // 2), jnp.float32),
            pltpu.VMEM((128, 128), jnp.float32),
            pltpu.VMEM((128, 128), jnp.float32),
            pltpu.VMEM((64, 128), jnp.float32),
            pltpu.VMEM((64, 128), jnp.float32),
            pltpu.VMEM((128,), jnp.int32),
            pltpu.VMEM((128,), jnp.int32),
            pltpu.VMEM((64,), jnp.int32),
            pltpu.VMEM((64,), jnp.int32),
            pltpu.VMEM((64,), jnp.int32),
            pltpu.VMEM((64,), jnp.int32),
            pltpu.SemaphoreType.DMA,
            pltpu.SemaphoreType.DMA,
            pltpu.SemaphoreType.DMA,
            pltpu.SemaphoreType.DMA,
        ],
    )(x_flat, cache2d, emb)


def kernel(x, cache, emb):
    out2d = _net(x.reshape(_S), cache.reshape(_C * _L // 128, 128), emb)
    return out2d.reshape(1, _C, 1, _L)
